# baseline (speedup 1.0000x reference)
import jax
import jax.numpy as jnp
from jax import lax
from jax.experimental import pallas as pl
from jax.experimental.pallas import tpu as pltpu
from jax.experimental.pallas import tpu_sc as plsc


_C = 64


def _sc_body(x_hbm, nl_hbm, out_hbm, buf, zbuf, nl_v, sem):
    H, twoB, D = x_hbm.shape
    B = twoB // 2
    C = _C
    nch = H // C

    c = lax.axis_index("c")
    s = lax.axis_index("s")
    b = s
    h = c
    src_col = h * B + b
    dst_col = b * 2 + h

    pltpu.sync_copy(nl_hbm, nl_v)
    nl = nl_v[pl.ds(b, 16)][0]

    zeros16 = jnp.zeros((16,), jnp.float32)
    nlanes = D // 16

    def zstore(ref, i):
        ref[i // nlanes, pl.ds((i % nlanes) * 16, 16)] = zeros16

    def zinit(i, carry):
        zstore(zbuf, i)
        return carry

    lax.fori_loop(0, C * nlanes, zinit, 0)

    def gather(t0):
        pltpu.make_async_copy(
            x_hbm.at[pl.ds(t0, C), src_col, :], buf, sem
        ).start()
        pltpu.make_async_copy(
            x_hbm.at[pl.ds(t0, C), src_col, :], buf, sem
        ).wait()

    def scatter(src, t0):
        pltpu.make_async_copy(
            src, out_hbm.at[pl.ds(t0, C), dst_col, :], sem
        ).start()
        pltpu.make_async_copy(
            src, out_hbm.at[pl.ds(t0, C), dst_col, :], sem
        ).wait()

    def step(k, carry):
        t0 = k * C
        full_valid = t0 + C <= nl
        full_invalid = t0 >= nl

        @pl.when(full_valid)
        def _():
            gather(t0)
            scatter(buf, t0)

        @pl.when(full_invalid)
        def _():
            scatter(zbuf, t0)

        @pl.when(jnp.logical_not(jnp.logical_or(full_valid, full_invalid)))
        def _():
            gather(t0)

            def ztail(i, carry):
                zstore(buf, i)
                return carry

            lax.fori_loop((nl - t0) * nlanes, C * nlanes, ztail, 0)
            scatter(buf, t0)

        return carry

    lax.fori_loop(0, nch, step, 0)


def kernel(data, lengths):
    T, B, D = data.shape
    T2 = T - (T % 2)
    H = T2 // 2
    newlens = (lengths // 2).astype(jnp.int32)
    nl_padded = jnp.concatenate([newlens, jnp.zeros((B,), jnp.int32)])
    x = data[:T2].reshape(H, 2 * B, D)

    sc_call = pl.kernel(
        _sc_body,
        out_type=jax.ShapeDtypeStruct((H, 2 * B, D), data.dtype),
        mesh=plsc.VectorSubcoreMesh(core_axis_name="c", subcore_axis_name="s"),
        scratch_types=[
            pltpu.VMEM((_C, D), jnp.float32),
            pltpu.VMEM((_C, D), jnp.float32),
            pltpu.VMEM((2 * B,), jnp.int32),
            pltpu.SemaphoreType.DMA,
        ],
    )
    out = sc_call(x, nl_padded)
    return out.reshape(H, B, 2 * D), newlens


# speedup vs baseline: 1.0780x; 1.0780x over previous
import jax
import jax.numpy as jnp
from jax import lax
from jax.experimental import pallas as pl
from jax.experimental.pallas import tpu as pltpu
from jax.experimental.pallas import tpu_sc as plsc


_C = 64


def _sc_body(x_hbm, nl_hbm, out_hbm, buf0, buf1, zbuf, nl_v, sem_g, sem_s, sem_z):
    H, twoB, D = x_hbm.shape
    B = twoB // 2
    C = _C
    nch = H // C

    c = lax.axis_index("c")
    s = lax.axis_index("s")
    b = s
    h = c
    src_col = h * B + b
    dst_col = b * 2 + h

    pltpu.sync_copy(nl_hbm, nl_v)
    nl = nl_v[pl.ds(b, 16)][0]
    kz = (nl + C - 1) // C

    zeros16 = jnp.zeros((16,), jnp.float32)
    nlanes = D // 16

    def zstore(ref, i):
        ref[i // nlanes, pl.ds((i % nlanes) * 16, 16)] = zeros16

    def zinit(i, carry):
        zstore(zbuf, i)
        return carry

    lax.fori_loop(0, C * nlanes, zinit, 0)

    def gather_cp(t0, buf):
        return pltpu.make_async_copy(
            x_hbm.at[pl.ds(t0, C), src_col, :], buf, sem_g
        )

    def scatter_cp(src, t0, sem):
        return pltpu.make_async_copy(
            src, out_hbm.at[pl.ds(t0, C), dst_col, :], sem
        )

    def zfire(k, carry):
        scatter_cp(zbuf, k * C, sem_z).start()
        return carry

    lax.fori_loop(kz, nch, zfire, 0)

    def per_buf(k, fn):
        @pl.when(k % 2 == 0)
        def _():
            fn(buf0)

        @pl.when(k % 2 == 1)
        def _():
            fn(buf1)

    @pl.when(kz > 0)
    def _():
        gather_cp(0, buf0).start()

    def step(k, carry):
        t0 = k * C
        gather_cp(t0, buf0).wait()

        @pl.when(k + 1 < kz)
        def _():
            per_buf(k + 1, lambda bf: gather_cp(t0 + C, bf).start())

        @pl.when(t0 + C > nl)
        def _():
            def ztail(i, carry):
                per_buf(k, lambda bf: zstore(bf, i))
                return carry

            lax.fori_loop((nl - t0) * nlanes, C * nlanes, ztail, 0)

        per_buf(k, lambda bf: scatter_cp(bf, t0, sem_s).start())
        per_buf(k, lambda bf: scatter_cp(bf, t0, sem_s).wait())
        return carry

    lax.fori_loop(0, kz, step, 0)

    def zdrain(k, carry):
        scatter_cp(zbuf, k * C, sem_z).wait()
        return carry

    lax.fori_loop(kz, nch, zdrain, 0)


def kernel(data, lengths):
    T, B, D = data.shape
    T2 = T - (T % 2)
    H = T2 // 2
    newlens = (lengths // 2).astype(jnp.int32)
    nl_padded = jnp.concatenate([newlens, jnp.zeros((B,), jnp.int32)])
    x = data[:T2].reshape(H, 2 * B, D)

    sc_call = pl.kernel(
        _sc_body,
        out_type=jax.ShapeDtypeStruct((H, 2 * B, D), data.dtype),
        mesh=plsc.VectorSubcoreMesh(core_axis_name="c", subcore_axis_name="s"),
        scratch_types=[
            pltpu.VMEM((_C, D), jnp.float32),
            pltpu.VMEM((_C, D), jnp.float32),
            pltpu.VMEM((_C, D), jnp.float32),
            pltpu.VMEM((2 * B,), jnp.int32),
            pltpu.SemaphoreType.DMA,
            pltpu.SemaphoreType.DMA,
            pltpu.SemaphoreType.DMA,
        ],
    )
    out = sc_call(x, nl_padded)
    return out.reshape(H, B, 2 * D), newlens


# speedup vs baseline: 4.6369x; 4.3015x over previous
import jax
import jax.numpy as jnp
from jax import lax
from jax.experimental import pallas as pl
from jax.experimental.pallas import tpu as pltpu


_G = 128


def _body(lens_sref, lens3_ref, x_any, out_ref, scratch, sem):
    k = pl.program_id(0)
    n = pl.num_programs(0)
    g, b_dim, two_d = out_ref.shape
    B = b_dim

    def fetch(kk, slot, op):
        for b in range(B):
            cond = jnp.logical_and(kk < n, kk * g < lens_sref[b])

            @pl.when(cond)
            def _():
                cp = pltpu.make_async_copy(
                    x_any.at[pl.ds(kk * g, g), :, b, :],
                    scratch.at[slot, :, :, b, :],
                    sem,
                )
                if op == "start":
                    cp.start()
                else:
                    cp.wait()

    @pl.when(k == 0)
    def _():
        fetch(0, 0, "start")

    fetch(k + 1, (k + 1) % 2, "start")
    fetch(k, k % 2, "wait")

    def compute(slot):
        merged = jnp.concatenate(
            [scratch[slot, :, 0], scratch[slot, :, 1]], axis=-1
        )
        t2 = k * g + lax.broadcasted_iota(jnp.int32, (g, B, two_d), 0)
        lens3 = jnp.broadcast_to(lens3_ref[...], (g, B, two_d))
        out_ref[...] = jnp.where(t2 < lens3, merged, 0.0)

    @pl.when(k % 2 == 0)
    def _():
        compute(0)

    @pl.when(k % 2 == 1)
    def _():
        compute(1)


def kernel(data, lengths):
    T, B, D = data.shape
    T2 = T - (T % 2)
    H = T2 // 2
    newlens = (lengths // 2).astype(jnp.int32)
    x = data[:T2].reshape(H, 2, B, D)
    lens3d = newlens.reshape(1, B, 1)

    grid_spec = pltpu.PrefetchScalarGridSpec(
        num_scalar_prefetch=1,
        grid=(H // _G,),
        in_specs=[
            pl.BlockSpec((1, B, 1), lambda k, lens: (0, 0, 0)),
            pl.BlockSpec(memory_space=pl.ANY),
        ],
        out_specs=pl.BlockSpec((_G, B, 2 * D), lambda k, lens: (k, 0, 0)),
        scratch_shapes=[
            pltpu.VMEM((2, _G, 2, B, D), jnp.float32),
            pltpu.SemaphoreType.DMA,
        ],
    )
    out = pl.pallas_call(
        _body,
        grid_spec=grid_spec,
        out_shape=jax.ShapeDtypeStruct((H, B, 2 * D), data.dtype),
    )(newlens, lens3d, x)
    return out, newlens
